# baseline (speedup 1.0000x reference)
import functools

import jax
import jax.numpy as jnp
from jax import lax
from jax.experimental import pallas as pl
from jax.experimental.pallas import tpu as pltpu
from jax.experimental.pallas import tpu_sc as plsc

NE = 16
NT = 4096
NA = 2 * NT
BLK = 256
NBMAX = NA // BLK + NE
CAP = NBMAX * BLK
CHUNK = 512
NCHUNK = NA // CHUNK
NC, NS = 2, 16
NW = NC * NS
TPW = NT // NW


def _routing_body(x_ref, wr_ref, br_ref, s0_ref, s1_ref, wf_ref, bexp_ref,
                  oh_ref, call_ref, p_ref):
    logits = (
        jnp.dot(x_ref[...], wr_ref[...], preferred_element_type=jnp.float32)
        + br_ref[...]
    )
    m = jnp.max(logits, axis=-1, keepdims=True)
    ex = jnp.exp(logits - m)
    probs = ex / jnp.sum(ex, axis=-1, keepdims=True)

    lane = lax.broadcasted_iota(jnp.int32, probs.shape, 1)
    m1 = jnp.max(probs, axis=-1, keepdims=True)
    a1 = jnp.min(jnp.where(probs == m1, lane, NE), axis=-1, keepdims=True)
    rest = jnp.where(lane == a1, -jnp.inf, probs)
    m2 = jnp.max(rest, axis=-1, keepdims=True)
    a2 = jnp.min(jnp.where(rest == m2, lane, NE), axis=-1, keepdims=True)
    wf_ref[0:NT] = m1
    wf_ref[NT:NA] = m2
    oh_ref[0:NT] = (lane == a1).astype(jnp.float32)
    oh_ref[NT:NA] = (lane == a2).astype(jnp.float32)

    r_i = lax.broadcasted_iota(jnp.int32, (CHUNK, CHUNK), 0)
    c_i = lax.broadcasted_iota(jnp.int32, (CHUNK, CHUNK), 1)
    tri = (r_i >= c_i).astype(jnp.float32)

    def chunk_cumsum(c, _):
        oc = oh_ref[pl.ds(c * CHUNK, CHUNK), :]
        cc = jnp.dot(tri, oc, preferred_element_type=jnp.float32)
        call_ref[pl.ds(c * CHUNK, CHUNK), :] = cc
        p_ref[pl.ds(c, 1), :] = cc[CHUNK - 1 : CHUNK, :]
        return 0

    lax.fori_loop(0, NCHUNK, chunk_cumsum, 0)

    rn = lax.broadcasted_iota(jnp.int32, (NCHUNK, NCHUNK), 0)
    cn = lax.broadcasted_iota(jnp.int32, (NCHUNK, NCHUNK), 1)
    trin = (rn > cn).astype(jnp.float32)
    totals = p_ref[...]
    pref = jnp.dot(trin, totals, preferred_element_type=jnp.float32)
    counts = pref[NCHUNK - 1 : NCHUNK, :] + totals[NCHUNK - 1 : NCHUNK, :]

    nb = jnp.floor((counts + (BLK - 1)) * (1.0 / BLK))
    le = lax.broadcasted_iota(jnp.int32, (NE, NE), 0)
    lf = lax.broadcasted_iota(jnp.int32, (NE, NE), 1)
    u_excl = (le < lf).astype(jnp.float32)
    u_incl = (le <= lf).astype(jnp.float32)
    start_rows = BLK * jnp.dot(nb, u_excl, preferred_element_type=jnp.float32)
    cum_incl = jnp.dot(nb, u_incl, preferred_element_type=jnp.float32)
    total_blocks = cum_incl[0:1, NE - 1 : NE]
    bi = lax.broadcasted_iota(jnp.int32, (NBMAX, NE), 0).astype(jnp.float32)
    bexp = jnp.sum((cum_incl <= bi).astype(jnp.float32), axis=-1, keepdims=True)
    act = bi[:, 0:1] < total_blocks
    bexp_ref[...] = jnp.where(act, bexp, NE - 1.0).astype(jnp.int32)

    p_ref[...] = pref

    def chunk_slot(c, _):
        oc = oh_ref[pl.ds(c * CHUNK, CHUNK), :]
        r_incl = call_ref[pl.ds(c * CHUNK, CHUNK), :] + p_ref[pl.ds(c, 1), :]
        r_excl = r_incl - oc
        slot = jnp.sum((start_rows + r_excl) * oc, axis=-1, keepdims=True)
        half = c < (NCHUNK // 2)
        slot_i = slot.astype(jnp.int32)

        @pl.when(half)
        def _():
            s0_ref[pl.ds(c * CHUNK, CHUNK), :] = slot_i

        @pl.when(jnp.logical_not(half))
        def _():
            s1_ref[pl.ds(c * CHUNK - NT, CHUNK), :] = slot_i

        return 0

    lax.fori_loop(0, NCHUNK, chunk_slot, 0)


def _routing(x, Wr, br):
    n, d = x.shape
    return pl.pallas_call(
        _routing_body,
        in_specs=[
            pl.BlockSpec((n, d), lambda: (0, 0)),
            pl.BlockSpec((d, NE), lambda: (0, 0)),
            pl.BlockSpec((1, NE), lambda: (0, 0)),
        ],
        out_specs=[
            pl.BlockSpec((NT, 1), lambda: (0, 0)),
            pl.BlockSpec((NT, 1), lambda: (0, 0)),
            pl.BlockSpec((NA, 1), lambda: (0, 0)),
            pl.BlockSpec((NBMAX, 1), lambda: (0, 0)),
        ],
        out_shape=[
            jax.ShapeDtypeStruct((NT, 1), jnp.int32),
            jax.ShapeDtypeStruct((NT, 1), jnp.int32),
            jax.ShapeDtypeStruct((NA, 1), jnp.float32),
            jax.ShapeDtypeStruct((NBMAX, 1), jnp.int32),
        ],
        scratch_shapes=[
            pltpu.VMEM((NA, NE), jnp.float32),
            pltpu.VMEM((NA, NE), jnp.float32),
            pltpu.VMEM((NCHUNK, NE), jnp.float32),
        ],
    )(x, Wr, br.reshape(1, NE))


def _dispatch_body(
    x_hbm, s0_hbm, s1_hbm, wf_hbm, xs_hbm, ws_hbm,
    rows_a, rows_b, i0_v, i1_v, w0_v, w1_v, sem_ld, sem_st,
):
    wid = lax.axis_index("s") * NC + lax.axis_index("c")
    rows_per = 32
    nch = TPW // rows_per
    bufs = (rows_a, rows_b)

    base0 = wid * TPW
    ld0 = pltpu.async_copy(x_hbm.at[pl.ds(base0, rows_per)], rows_a, sem_ld)
    for ch in range(nch):
        base = wid * TPW + ch * rows_per
        cur = bufs[ch % 2]
        pltpu.sync_copy(s0_hbm.at[pl.ds(base, rows_per)], i0_v)
        pltpu.sync_copy(s1_hbm.at[pl.ds(base, rows_per)], i1_v)
        pltpu.sync_copy(wf_hbm.at[pl.ds(base, rows_per)], w0_v)
        pltpu.sync_copy(wf_hbm.at[pl.ds(NT + base, rows_per)], w1_v)
        if ch == 0:
            ld0.wait()
        else:
            ld.wait()
        if ch + 1 < nch:
            nxt = bufs[(ch + 1) % 2]
            ld = pltpu.async_copy(
                x_hbm.at[pl.ds(base + rows_per, rows_per)], nxt, sem_ld
            )
        pltpu.async_copy(cur, xs_hbm.at[i0_v], sem_st).wait()
        pltpu.async_copy(cur, xs_hbm.at[i1_v], sem_st).wait()
        pltpu.async_copy(w0_v, ws_hbm.at[i0_v], sem_st).wait()
        pltpu.async_copy(w1_v, ws_hbm.at[i1_v], sem_st).wait()


def _dispatch(x, s0, s1, wf):
    d = x.shape[1]
    mesh = plsc.VectorSubcoreMesh(
        core_axis_name="c", subcore_axis_name="s", num_cores=NC, num_subcores=NS
    )
    f = functools.partial(
        pl.kernel,
        out_type=(
            jax.ShapeDtypeStruct((CAP, d), jnp.float32),
            jax.ShapeDtypeStruct((CAP,), jnp.float32),
        ),
        mesh=mesh,
        scratch_types=[
            pltpu.VMEM((32, d), jnp.float32),
            pltpu.VMEM((32, d), jnp.float32),
            pltpu.VMEM((32,), jnp.int32),
            pltpu.VMEM((32,), jnp.int32),
            pltpu.VMEM((32,), jnp.float32),
            pltpu.VMEM((32,), jnp.float32),
            pltpu.SemaphoreType.DMA,
            pltpu.SemaphoreType.DMA,
        ],
    )(_dispatch_body)
    return f(x, s0, s1, wf)


def _gmm_body(bexp_smem, x_ref, we_ref, be_ref, ws_ref, y_ref):
    del bexp_smem
    y = jnp.dot(x_ref[...], we_ref[0], preferred_element_type=jnp.float32)
    y_ref[...] = jnp.maximum(y + be_ref[0], 0.0) * ws_ref[...]


def _gmm(block_expert, xs, ws, We, be):
    ne, d, u = We.shape
    grid_spec = pltpu.PrefetchScalarGridSpec(
        num_scalar_prefetch=1,
        grid=(NBMAX,),
        in_specs=[
            pl.BlockSpec((BLK, d), lambda b, bexp: (b, 0)),
            pl.BlockSpec((1, d, u), lambda b, bexp: (bexp[b], 0, 0)),
            pl.BlockSpec((1, 1, u), lambda b, bexp: (bexp[b], 0, 0)),
            pl.BlockSpec((BLK, 1), lambda b, bexp: (b, 0)),
        ],
        out_specs=pl.BlockSpec((BLK, u), lambda b, bexp: (b, 0)),
    )
    return pl.pallas_call(
        _gmm_body,
        grid_spec=grid_spec,
        out_shape=jax.ShapeDtypeStruct((CAP, u), jnp.float32),
        compiler_params=pltpu.CompilerParams(dimension_semantics=("arbitrary",)),
    )(block_expert, xs, We, be.reshape(ne, 1, u), ws.reshape(CAP, 1))


def _combine_body(
    y_hbm, s0_hbm, s1_hbm, out_hbm,
    r0_a, r1_a, o_a, r0_b, r1_b, o_b, i0_a, i1_a, i0_b, i1_b,
    sem_a, sem_b, sem_o,
):
    wid = lax.axis_index("s") * NC + lax.axis_index("c")
    tpc = 16
    nch = TPW // tpc
    d = 1024
    r0s = (r0_a, r0_b)
    r1s = (r1_a, r1_b)
    os_ = (o_a, o_b)
    i0s = (i0_a, i0_b)
    i1s = (i1_a, i1_b)
    sems = (sem_a, sem_b)

    def issue(ch):
        base = wid * TPW + ch * tpc
        k = ch % 2
        pltpu.sync_copy(s0_hbm.at[pl.ds(base, tpc)], i0s[k])
        pltpu.sync_copy(s1_hbm.at[pl.ds(base, tpc)], i1s[k])
        g0 = pltpu.async_copy(y_hbm.at[i0s[k]], r0s[k], sems[k])
        g1 = pltpu.async_copy(y_hbm.at[i1s[k]], r1s[k], sems[k])
        return (g0, g1)

    pend = issue(0)
    st = None
    for ch in range(nch):
        k = ch % 2
        pend[0].wait()
        pend[1].wait()
        if ch + 1 < nch:
            pend = issue(ch + 1)
        if st is not None:
            st.wait()
        r0, r1, o = r0s[k], r1s[k], os_[k]

        def tok(i, _):
            def vec(v, _):
                sl = pl.ds(v * 16, 16)
                o[i, sl] = r0[i, sl] + r1[i, sl]
                return 0

            lax.fori_loop(0, d // 16, vec, 0, unroll=8)
            return 0

        lax.fori_loop(0, tpc, tok, 0)
        base = wid * TPW + ch * tpc
        st = pltpu.async_copy(o, out_hbm.at[pl.ds(base, tpc)], sem_o)
    st.wait()


def _combine(y, s0, s1):
    u = y.shape[1]
    mesh = plsc.VectorSubcoreMesh(
        core_axis_name="c", subcore_axis_name="s", num_cores=NC, num_subcores=NS
    )
    f = functools.partial(
        pl.kernel,
        out_type=jax.ShapeDtypeStruct((NT, u), jnp.float32),
        mesh=mesh,
        scratch_types=[
            pltpu.VMEM((16, u), jnp.float32),
            pltpu.VMEM((16, u), jnp.float32),
            pltpu.VMEM((16, u), jnp.float32),
            pltpu.VMEM((16, u), jnp.float32),
            pltpu.VMEM((16, u), jnp.float32),
            pltpu.VMEM((16, u), jnp.float32),
            pltpu.VMEM((16,), jnp.int32),
            pltpu.VMEM((16,), jnp.int32),
            pltpu.VMEM((16,), jnp.int32),
            pltpu.VMEM((16,), jnp.int32),
            pltpu.SemaphoreType.DMA,
            pltpu.SemaphoreType.DMA,
            pltpu.SemaphoreType.DMA,
        ],
    )(_combine_body)
    return f(y, s0, s1)


def kernel(inputs, Wr, br, We, be):
    s0c, s1c, wfc, bexpc = _routing(inputs, Wr, br)
    s0 = s0c.reshape(NT)
    s1 = s1c.reshape(NT)
    wf = wfc.reshape(NA)
    xs, ws = _dispatch(inputs, s0, s1, wf)
    y = _gmm(bexpc[:, 0], xs, ws, We, be)
    return _combine(y, s0, s1)


# speedup vs baseline: 1.0055x; 1.0055x over previous
import functools

import jax
import jax.numpy as jnp
from jax import lax
from jax.experimental import pallas as pl
from jax.experimental.pallas import tpu as pltpu
from jax.experimental.pallas import tpu_sc as plsc

NE = 16
NT = 4096
NA = 2 * NT
BLK = 256
NBMAX = NA // BLK + NE
CAP = NBMAX * BLK
CHUNK = 512
NCHUNK = NA // CHUNK
NC, NS = 2, 16
NW = NC * NS
TPW = NT // NW
TB = 1024
RPC = 32
NDCH = TPW // RPC
TPC = 16
NCCH = TPW // TPC


def _routing_body(x_ref, wr_ref, br_ref, s0_ref, s1_ref, wf_ref, bexp_ref,
                  probs_ref, oh_ref, call_ref, p_ref):
    t = pl.program_id(0)
    logits = (
        jnp.dot(x_ref[...], wr_ref[...], preferred_element_type=jnp.float32)
        + br_ref[...]
    )
    m = jnp.max(logits, axis=-1, keepdims=True)
    ex = jnp.exp(logits - m)
    probs_ref[pl.ds(t * TB, TB), :] = ex / jnp.sum(ex, axis=-1, keepdims=True)

    @pl.when(t == NT // TB - 1)
    def _routing_tail():
        probs = probs_ref[...]
        lane = lax.broadcasted_iota(jnp.int32, probs.shape, 1)
        m1 = jnp.max(probs, axis=-1, keepdims=True)
        a1 = jnp.min(jnp.where(probs == m1, lane, NE), axis=-1, keepdims=True)
        rest = jnp.where(lane == a1, -jnp.inf, probs)
        m2 = jnp.max(rest, axis=-1, keepdims=True)
        a2 = jnp.min(jnp.where(rest == m2, lane, NE), axis=-1, keepdims=True)
        wf_ref[0:NT] = m1
        wf_ref[NT:NA] = m2
        oh_ref[0:NT] = (lane == a1).astype(jnp.float32)
        oh_ref[NT:NA] = (lane == a2).astype(jnp.float32)

        r_i = lax.broadcasted_iota(jnp.int32, (CHUNK, CHUNK), 0)
        c_i = lax.broadcasted_iota(jnp.int32, (CHUNK, CHUNK), 1)
        tri = (r_i >= c_i).astype(jnp.float32)

        def chunk_cumsum(c, _):
            oc = oh_ref[pl.ds(c * CHUNK, CHUNK), :]
            cc = jnp.dot(tri, oc, preferred_element_type=jnp.float32)
            call_ref[pl.ds(c * CHUNK, CHUNK), :] = cc
            p_ref[pl.ds(c, 1), :] = cc[CHUNK - 1 : CHUNK, :]
            return 0

        lax.fori_loop(0, NCHUNK, chunk_cumsum, 0)

        rn = lax.broadcasted_iota(jnp.int32, (NCHUNK, NCHUNK), 0)
        cn = lax.broadcasted_iota(jnp.int32, (NCHUNK, NCHUNK), 1)
        trin = (rn > cn).astype(jnp.float32)
        totals = p_ref[...]
        pref = jnp.dot(trin, totals, preferred_element_type=jnp.float32)
        counts = pref[NCHUNK - 1 : NCHUNK, :] + totals[NCHUNK - 1 : NCHUNK, :]

        nb = jnp.floor((counts + (BLK - 1)) * (1.0 / BLK))
        le = lax.broadcasted_iota(jnp.int32, (NE, NE), 0)
        lf = lax.broadcasted_iota(jnp.int32, (NE, NE), 1)
        u_excl = (le < lf).astype(jnp.float32)
        u_incl = (le <= lf).astype(jnp.float32)
        start_rows = BLK * jnp.dot(nb, u_excl, preferred_element_type=jnp.float32)
        cum_incl = jnp.dot(nb, u_incl, preferred_element_type=jnp.float32)
        total_blocks = cum_incl[0:1, NE - 1 : NE]
        bi = lax.broadcasted_iota(jnp.int32, (NBMAX, NE), 0).astype(jnp.float32)
        bexp = jnp.sum((cum_incl <= bi).astype(jnp.float32), axis=-1, keepdims=True)
        act = bi[:, 0:1] < total_blocks
        bexp_ref[...] = jnp.where(act, bexp, NE - 1.0).astype(jnp.int32)

        p_ref[...] = pref

        def chunk_slot(c, _):
            oc = oh_ref[pl.ds(c * CHUNK, CHUNK), :]
            r_incl = call_ref[pl.ds(c * CHUNK, CHUNK), :] + p_ref[pl.ds(c, 1), :]
            r_excl = r_incl - oc
            slot = jnp.sum((start_rows + r_excl) * oc, axis=-1, keepdims=True)
            half = c < (NCHUNK // 2)
            slot_i = slot.astype(jnp.int32)

            @pl.when(half)
            def _():
                s0_ref[pl.ds(c * CHUNK, CHUNK), :] = slot_i

            @pl.when(jnp.logical_not(half))
            def _():
                s1_ref[pl.ds(c * CHUNK - NT, CHUNK), :] = slot_i

            return 0

        lax.fori_loop(0, NCHUNK, chunk_slot, 0)


def _routing(x, Wr, br):
    n, d = x.shape
    return pl.pallas_call(
        _routing_body,
        grid=(n // TB,),
        in_specs=[
            pl.BlockSpec((TB, d), lambda t: (t, 0)),
            pl.BlockSpec((d, NE), lambda t: (0, 0)),
            pl.BlockSpec((1, NE), lambda t: (0, 0)),
        ],
        out_specs=[
            pl.BlockSpec((NT, 1), lambda t: (0, 0)),
            pl.BlockSpec((NT, 1), lambda t: (0, 0)),
            pl.BlockSpec((NA, 1), lambda t: (0, 0)),
            pl.BlockSpec((NBMAX, 1), lambda t: (0, 0)),
        ],
        out_shape=[
            jax.ShapeDtypeStruct((NT, 1), jnp.int32),
            jax.ShapeDtypeStruct((NT, 1), jnp.int32),
            jax.ShapeDtypeStruct((NA, 1), jnp.float32),
            jax.ShapeDtypeStruct((NBMAX, 1), jnp.int32),
        ],
        scratch_shapes=[
            pltpu.VMEM((NT, NE), jnp.float32),
            pltpu.VMEM((NA, NE), jnp.float32),
            pltpu.VMEM((NA, NE), jnp.float32),
            pltpu.VMEM((NCHUNK, NE), jnp.float32),
        ],
        compiler_params=pltpu.CompilerParams(dimension_semantics=("arbitrary",)),
    )(x, Wr, br.reshape(1, NE))


def _dispatch_body(
    x_hbm, s0_hbm, s1_hbm, w0_hbm, w1_hbm, xs_hbm, ws_hbm,
    rows_a, rows_b, i0_s, i1_s, w0_s, w1_s, sem_ld, sem_st,
):
    wid = lax.axis_index("s") * NC + lax.axis_index("c")
    bufs = (rows_a, rows_b)
    pltpu.sync_copy(s0_hbm.at[wid], i0_s)
    pltpu.sync_copy(s1_hbm.at[wid], i1_s)
    pltpu.sync_copy(w0_hbm.at[wid], w0_s)
    pltpu.sync_copy(w1_hbm.at[wid], w1_s)

    base0 = wid * TPW
    ld = pltpu.async_copy(x_hbm.at[pl.ds(base0, RPC)], rows_a, sem_ld)
    for ch in range(NDCH):
        cur = bufs[ch % 2]
        ld.wait()
        if ch + 1 < NDCH:
            ld = pltpu.async_copy(
                x_hbm.at[pl.ds(base0 + (ch + 1) * RPC, RPC)], bufs[(ch + 1) % 2],
                sem_ld,
            )
        h0 = pltpu.async_copy(cur, xs_hbm.at[i0_s.at[ch]], sem_st)
        h1 = pltpu.async_copy(cur, xs_hbm.at[i1_s.at[ch]], sem_st)
        h2 = pltpu.async_copy(w0_s.at[ch], ws_hbm.at[i0_s.at[ch]], sem_st)
        h3 = pltpu.async_copy(w1_s.at[ch], ws_hbm.at[i1_s.at[ch]], sem_st)
        h0.wait()
        h1.wait()
        h2.wait()
        h3.wait()


def _dispatch(x, s0_3d, s1_3d, w0_3d, w1_3d):
    d = x.shape[1]
    mesh = plsc.VectorSubcoreMesh(
        core_axis_name="c", subcore_axis_name="s", num_cores=NC, num_subcores=NS
    )
    f = functools.partial(
        pl.kernel,
        out_type=(
            jax.ShapeDtypeStruct((CAP, d), jnp.float32),
            jax.ShapeDtypeStruct((CAP,), jnp.float32),
        ),
        mesh=mesh,
        scratch_types=[
            pltpu.VMEM((RPC, d), jnp.float32),
            pltpu.VMEM((RPC, d), jnp.float32),
            pltpu.VMEM((NDCH, RPC), jnp.int32),
            pltpu.VMEM((NDCH, RPC), jnp.int32),
            pltpu.VMEM((NDCH, RPC), jnp.float32),
            pltpu.VMEM((NDCH, RPC), jnp.float32),
            pltpu.SemaphoreType.DMA,
            pltpu.SemaphoreType.DMA,
        ],
    )(_dispatch_body)
    return f(x, s0_3d, s1_3d, w0_3d, w1_3d)


def _gmm_body(bexp_smem, x_ref, we_ref, be_ref, ws_ref, y_ref):
    del bexp_smem
    y = jnp.dot(x_ref[...], we_ref[0], preferred_element_type=jnp.float32)
    y_ref[...] = jnp.maximum(y + be_ref[0], 0.0) * ws_ref[...]


def _gmm(block_expert, xs, ws, We, be):
    ne, d, u = We.shape
    grid_spec = pltpu.PrefetchScalarGridSpec(
        num_scalar_prefetch=1,
        grid=(NBMAX,),
        in_specs=[
            pl.BlockSpec((BLK, d), lambda b, bexp: (b, 0)),
            pl.BlockSpec((1, d, u), lambda b, bexp: (bexp[b], 0, 0)),
            pl.BlockSpec((1, 1, u), lambda b, bexp: (bexp[b], 0, 0)),
            pl.BlockSpec((BLK, 1), lambda b, bexp: (b, 0)),
        ],
        out_specs=pl.BlockSpec((BLK, u), lambda b, bexp: (b, 0)),
    )
    return pl.pallas_call(
        _gmm_body,
        grid_spec=grid_spec,
        out_shape=jax.ShapeDtypeStruct((CAP, u), jnp.float32),
        compiler_params=pltpu.CompilerParams(dimension_semantics=("arbitrary",)),
    )(block_expert, xs, We, be.reshape(ne, 1, u), ws.reshape(CAP, 1))


def _combine_body(
    y_hbm, s0_hbm, s1_hbm, out_hbm,
    r0_a, r1_a, o_a, r0_b, r1_b, o_b, i0_all, i1_all,
    sem_a, sem_b, sem_o,
):
    wid = lax.axis_index("s") * NC + lax.axis_index("c")
    d = 1024
    r0s = (r0_a, r0_b)
    r1s = (r1_a, r1_b)
    os_ = (o_a, o_b)
    sems = (sem_a, sem_b)
    pltpu.sync_copy(s0_hbm.at[wid], i0_all)
    pltpu.sync_copy(s1_hbm.at[wid], i1_all)

    def issue(ch):
        k = ch % 2
        idx0 = i0_all[pl.ds(ch * TPC, TPC)]
        idx1 = i1_all[pl.ds(ch * TPC, TPC)]
        g0 = pltpu.async_copy(y_hbm.at[idx0], r0s[k], sems[k])
        g1 = pltpu.async_copy(y_hbm.at[idx1], r1s[k], sems[k])
        return (g0, g1)

    pend = issue(0)
    st = None
    for ch in range(NCCH):
        k = ch % 2
        pend[0].wait()
        pend[1].wait()
        if ch + 1 < NCCH:
            pend = issue(ch + 1)
        if st is not None:
            st.wait()
        r0, r1, o = r0s[k], r1s[k], os_[k]

        def tok(i, _):
            def vec(v, _):
                sl = pl.ds(v * 16, 16)
                o[i, sl] = r0[i, sl] + r1[i, sl]
                return 0

            lax.fori_loop(0, d // 16, vec, 0, unroll=8)
            return 0

        lax.fori_loop(0, TPC, tok, 0)
        base = wid * TPW + ch * TPC
        st = pltpu.async_copy(o, out_hbm.at[pl.ds(base, TPC)], sem_o)
    st.wait()


def _combine(y, s0_2d, s1_2d):
    u = y.shape[1]
    mesh = plsc.VectorSubcoreMesh(
        core_axis_name="c", subcore_axis_name="s", num_cores=NC, num_subcores=NS
    )
    f = functools.partial(
        pl.kernel,
        out_type=jax.ShapeDtypeStruct((NT, u), jnp.float32),
        mesh=mesh,
        scratch_types=[
            pltpu.VMEM((TPC, u), jnp.float32),
            pltpu.VMEM((TPC, u), jnp.float32),
            pltpu.VMEM((TPC, u), jnp.float32),
            pltpu.VMEM((TPC, u), jnp.float32),
            pltpu.VMEM((TPC, u), jnp.float32),
            pltpu.VMEM((TPC, u), jnp.float32),
            pltpu.VMEM((TPW,), jnp.int32),
            pltpu.VMEM((TPW,), jnp.int32),
            pltpu.SemaphoreType.DMA,
            pltpu.SemaphoreType.DMA,
            pltpu.SemaphoreType.DMA,
        ],
    )(_combine_body)
    return f(y, s0_2d, s1_2d)


def kernel(inputs, Wr, br, We, be):
    s0c, s1c, wfc, bexpc = _routing(inputs, Wr, br)
    s0 = s0c.reshape(NT)
    s1 = s1c.reshape(NT)
    w0 = wfc[:NT].reshape(NW, NDCH, RPC)
    w1 = wfc[NT:].reshape(NW, NDCH, RPC)
    xs, ws = _dispatch(
        inputs,
        s0.reshape(NW, NDCH, RPC),
        s1.reshape(NW, NDCH, RPC),
        w0,
        w1,
    )
    y = _gmm(bexpc[:, 0], xs, ws, We, be)
    return _combine(y, s0.reshape(NW, TPW), s1.reshape(NW, TPW))


# speedup vs baseline: 1.1347x; 1.1285x over previous
import functools

import jax
import jax.numpy as jnp
from jax import lax
from jax.experimental import pallas as pl
from jax.experimental.pallas import tpu as pltpu
from jax.experimental.pallas import tpu_sc as plsc

NE = 16
NT = 4096
NA = 2 * NT
BLK = 256
NBMAX = NA // BLK + NE
CAP = NBMAX * BLK
CHUNK = 512
NCHUNK = NA // CHUNK
NC, NS = 2, 16
NW = NC * NS
TPW = NT // NW
TB = 1024
RPC = 32
NDCH = TPW // RPC
TPC = 16
NCCH = TPW // TPC


def _routing_body(x_ref, wr_ref, br_ref, s0_ref, s1_ref, w_ref,
                  bexp_ref, probs_ref, oh_ref, call_ref, p_ref):
    t = pl.program_id(0)
    xb = x_ref[...]
    logits = (
        jnp.dot(xb, wr_ref[...], preferred_element_type=jnp.float32)
        + br_ref[...]
    )
    m = jnp.max(logits, axis=-1, keepdims=True)
    ex = jnp.exp(logits - m)
    probs_ref[pl.ds(t * TB, TB), :] = ex / jnp.sum(ex, axis=-1, keepdims=True)

    @pl.when(t == NT // TB - 1)
    def _routing_tail():
        probs = probs_ref[...]
        lane = lax.broadcasted_iota(jnp.int32, probs.shape, 1)
        m1 = jnp.max(probs, axis=-1, keepdims=True)
        a1 = jnp.min(jnp.where(probs == m1, lane, NE), axis=-1, keepdims=True)
        rest = jnp.where(lane == a1, -jnp.inf, probs)
        m2 = jnp.max(rest, axis=-1, keepdims=True)
        a2 = jnp.min(jnp.where(rest == m2, lane, NE), axis=-1, keepdims=True)
        w_ref[:, 0:1] = m1
        w_ref[:, 1:2] = m2
        oh_ref[0:NT] = (lane == a1).astype(jnp.float32)
        oh_ref[NT:NA] = (lane == a2).astype(jnp.float32)

        r_i = lax.broadcasted_iota(jnp.int32, (CHUNK, CHUNK), 0)
        c_i = lax.broadcasted_iota(jnp.int32, (CHUNK, CHUNK), 1)
        tri = (r_i >= c_i).astype(jnp.float32)

        def chunk_cumsum(c, _):
            oc = oh_ref[pl.ds(c * CHUNK, CHUNK), :]
            cc = jnp.dot(tri, oc, preferred_element_type=jnp.float32)
            call_ref[pl.ds(c * CHUNK, CHUNK), :] = cc
            p_ref[pl.ds(c, 1), :] = cc[CHUNK - 1 : CHUNK, :]
            return 0

        lax.fori_loop(0, NCHUNK, chunk_cumsum, 0)

        rn = lax.broadcasted_iota(jnp.int32, (NCHUNK, NCHUNK), 0)
        cn = lax.broadcasted_iota(jnp.int32, (NCHUNK, NCHUNK), 1)
        trin = (rn > cn).astype(jnp.float32)
        totals = p_ref[...]
        pref = jnp.dot(trin, totals, preferred_element_type=jnp.float32)
        counts = pref[NCHUNK - 1 : NCHUNK, :] + totals[NCHUNK - 1 : NCHUNK, :]

        nb = jnp.floor((counts + (BLK - 1)) * (1.0 / BLK))
        le = lax.broadcasted_iota(jnp.int32, (NE, NE), 0)
        lf = lax.broadcasted_iota(jnp.int32, (NE, NE), 1)
        u_excl = (le < lf).astype(jnp.float32)
        u_incl = (le <= lf).astype(jnp.float32)
        start_rows = BLK * jnp.dot(nb, u_excl, preferred_element_type=jnp.float32)
        cum_incl = jnp.dot(nb, u_incl, preferred_element_type=jnp.float32)
        total_blocks = cum_incl[0:1, NE - 1 : NE]
        bi = lax.broadcasted_iota(jnp.int32, (NBMAX, NE), 0).astype(jnp.float32)
        bexp = jnp.sum((cum_incl <= bi).astype(jnp.float32), axis=-1, keepdims=True)
        act = bi[:, 0:1] < total_blocks
        bexp_ref[...] = jnp.where(act, bexp, NE - 1.0).astype(jnp.int32)

        p_ref[...] = pref

        def chunk_slot(c, _):
            oc = oh_ref[pl.ds(c * CHUNK, CHUNK), :]
            r_incl = call_ref[pl.ds(c * CHUNK, CHUNK), :] + p_ref[pl.ds(c, 1), :]
            r_excl = r_incl - oc
            slot = jnp.sum((start_rows + r_excl) * oc, axis=-1, keepdims=True)
            half = c < (NCHUNK // 2)
            slot_i = slot.astype(jnp.int32)

            @pl.when(half)
            def _():
                s0_ref[pl.ds(c * CHUNK, CHUNK), :] = slot_i

            @pl.when(jnp.logical_not(half))
            def _():
                s1_ref[pl.ds(c * CHUNK - NT, CHUNK), :] = slot_i

            return 0

        lax.fori_loop(0, NCHUNK, chunk_slot, 0)


def _routing(x, Wr, br):
    n, d = x.shape
    return pl.pallas_call(
        _routing_body,
        grid=(n // TB,),
        in_specs=[
            pl.BlockSpec((TB, d), lambda t: (t, 0)),
            pl.BlockSpec((d, NE), lambda t: (0, 0)),
            pl.BlockSpec((1, NE), lambda t: (0, 0)),
        ],
        out_specs=[
            pl.BlockSpec((NT, 1), lambda t: (0, 0)),
            pl.BlockSpec((NT, 1), lambda t: (0, 0)),
            pl.BlockSpec((NT, 2), lambda t: (0, 0)),
            pl.BlockSpec((NBMAX, 1), lambda t: (0, 0)),
        ],
        out_shape=[
            jax.ShapeDtypeStruct((NT, 1), jnp.int32),
            jax.ShapeDtypeStruct((NT, 1), jnp.int32),
            jax.ShapeDtypeStruct((NT, 2), jnp.float32),
            jax.ShapeDtypeStruct((NBMAX, 1), jnp.int32),
        ],
        scratch_shapes=[
            pltpu.VMEM((NT, NE), jnp.float32),
            pltpu.VMEM((NA, NE), jnp.float32),
            pltpu.VMEM((NA, NE), jnp.float32),
            pltpu.VMEM((NCHUNK, NE), jnp.float32),
        ],
        compiler_params=pltpu.CompilerParams(dimension_semantics=("arbitrary",)),
    )(x, Wr, br.reshape(1, NE))


def _dispatch_body(
    x_hbm, s0_hbm, s1_hbm, xs_hbm,
    rows_a, rows_b, i0_s, i1_s, sem_ld, sem_st,
):
    wid = lax.axis_index("s") * NC + lax.axis_index("c")
    bufs = (rows_a, rows_b)
    pltpu.sync_copy(s0_hbm.at[wid], i0_s)
    pltpu.sync_copy(s1_hbm.at[wid], i1_s)

    base0 = wid * TPW
    ld = pltpu.async_copy(x_hbm.at[pl.ds(base0, RPC)], rows_a, sem_ld)
    for ch in range(NDCH):
        cur = bufs[ch % 2]
        ld.wait()
        if ch + 1 < NDCH:
            ld = pltpu.async_copy(
                x_hbm.at[pl.ds(base0 + (ch + 1) * RPC, RPC)], bufs[(ch + 1) % 2],
                sem_ld,
            )
        h0 = pltpu.async_copy(cur, xs_hbm.at[i0_s.at[ch]], sem_st)
        h1 = pltpu.async_copy(cur, xs_hbm.at[i1_s.at[ch]], sem_st)
        h0.wait()
        h1.wait()


def _dispatch(x, s0_3d, s1_3d):
    d = x.shape[1]
    mesh = plsc.VectorSubcoreMesh(
        core_axis_name="c", subcore_axis_name="s", num_cores=NC, num_subcores=NS
    )
    f = functools.partial(
        pl.kernel,
        out_type=jax.ShapeDtypeStruct((CAP, d), jnp.float32),
        mesh=mesh,
        scratch_types=[
            pltpu.VMEM((RPC, d), jnp.float32),
            pltpu.VMEM((RPC, d), jnp.float32),
            pltpu.VMEM((NDCH, RPC), jnp.int32),
            pltpu.VMEM((NDCH, RPC), jnp.int32),
            pltpu.SemaphoreType.DMA,
            pltpu.SemaphoreType.DMA,
        ],
    )(_dispatch_body)
    return f(x, s0_3d, s1_3d)


def _gmm_body(bexp_smem, x_ref, we_ref, be_ref, y_ref):
    del bexp_smem
    y = jnp.dot(x_ref[...], we_ref[0], preferred_element_type=jnp.float32)
    y_ref[...] = jnp.maximum(y + be_ref[0], 0.0)


def _gmm(block_expert, xs, We, be):
    ne, d, u = We.shape
    grid_spec = pltpu.PrefetchScalarGridSpec(
        num_scalar_prefetch=1,
        grid=(NBMAX,),
        in_specs=[
            pl.BlockSpec((BLK, d), lambda b, bexp: (b, 0)),
            pl.BlockSpec((1, d, u), lambda b, bexp: (bexp[b], 0, 0)),
            pl.BlockSpec((1, 1, u), lambda b, bexp: (bexp[b], 0, 0)),
        ],
        out_specs=pl.BlockSpec((BLK, u), lambda b, bexp: (b, 0)),
    )
    return pl.pallas_call(
        _gmm_body,
        grid_spec=grid_spec,
        out_shape=jax.ShapeDtypeStruct((CAP, u), jnp.float32),
        compiler_params=pltpu.CompilerParams(dimension_semantics=("arbitrary",)),
    )(block_expert, xs, We, be.reshape(ne, 1, u))


def _combine_body(
    y_hbm, s0_hbm, s1_hbm, w0_hbm, w1_hbm, out_hbm,
    r0_a, r1_a, o_a, r0_b, r1_b, o_b, i0_all, i1_all, w0_all, w1_all,
    sem_a, sem_b, sem_o,
):
    wid = lax.axis_index("s") * NC + lax.axis_index("c")
    d = 1024
    r0s = (r0_a, r0_b)
    r1s = (r1_a, r1_b)
    os_ = (o_a, o_b)
    sems = (sem_a, sem_b)
    pltpu.sync_copy(s0_hbm.at[wid], i0_all)
    pltpu.sync_copy(s1_hbm.at[wid], i1_all)
    pltpu.sync_copy(w0_hbm.at[wid], w0_all)
    pltpu.sync_copy(w1_hbm.at[wid], w1_all)

    def issue(ch):
        k = ch % 2
        idx0 = i0_all[pl.ds(ch * TPC, TPC)]
        idx1 = i1_all[pl.ds(ch * TPC, TPC)]
        g0 = pltpu.async_copy(y_hbm.at[idx0], r0s[k], sems[k])
        g1 = pltpu.async_copy(y_hbm.at[idx1], r1s[k], sems[k])
        return (g0, g1)

    pend = issue(0)
    st = None
    for ch in range(NCCH):
        k = ch % 2
        pend[0].wait()
        pend[1].wait()
        if ch + 1 < NCCH:
            pend = issue(ch + 1)
        if st is not None:
            st.wait()
        r0, r1, o = r0s[k], r1s[k], os_[k]
        wv0 = w0_all[pl.ds(ch * TPC, TPC)]
        wv1 = w1_all[pl.ds(ch * TPC, TPC)]

        def tok(i, _):
            i_vec = lax.broadcast_in_dim(i, (16,), ())
            wa = wv0.at[i_vec].get(mode="promise_in_bounds")
            wb = wv1.at[i_vec].get(mode="promise_in_bounds")

            def vec(v, _):
                sl = pl.ds(v * 16, 16)
                o[i, sl] = wa * r0[i, sl] + wb * r1[i, sl]
                return 0

            lax.fori_loop(0, d // 16, vec, 0, unroll=8)
            return 0

        lax.fori_loop(0, TPC, tok, 0)
        base = wid * TPW + ch * TPC
        st = pltpu.async_copy(o, out_hbm.at[pl.ds(base, TPC)], sem_o)
    st.wait()


def _combine(y, s0_2d, s1_2d, w0_2d, w1_2d):
    u = y.shape[1]
    mesh = plsc.VectorSubcoreMesh(
        core_axis_name="c", subcore_axis_name="s", num_cores=NC, num_subcores=NS
    )
    f = functools.partial(
        pl.kernel,
        out_type=jax.ShapeDtypeStruct((NT, u), jnp.float32),
        mesh=mesh,
        scratch_types=[
            pltpu.VMEM((TPC, u), jnp.float32),
            pltpu.VMEM((TPC, u), jnp.float32),
            pltpu.VMEM((TPC, u), jnp.float32),
            pltpu.VMEM((TPC, u), jnp.float32),
            pltpu.VMEM((TPC, u), jnp.float32),
            pltpu.VMEM((TPC, u), jnp.float32),
            pltpu.VMEM((TPW,), jnp.int32),
            pltpu.VMEM((TPW,), jnp.int32),
            pltpu.VMEM((TPW,), jnp.float32),
            pltpu.VMEM((TPW,), jnp.float32),
            pltpu.SemaphoreType.DMA,
            pltpu.SemaphoreType.DMA,
            pltpu.SemaphoreType.DMA,
        ],
    )(_combine_body)
    return f(y, s0_2d, s1_2d, w0_2d, w1_2d)


def kernel(inputs, Wr, br, We, be):
    s0c, s1c, w, bexpc = _routing(inputs, Wr, br)
    s0 = s0c.reshape(NT)
    s1 = s1c.reshape(NT)
    xs = _dispatch(inputs, s0.reshape(NW, NDCH, RPC), s1.reshape(NW, NDCH, RPC))
    y = _gmm(bexpc[:, 0], xs, We, be)
    return _combine(
        y,
        s0.reshape(NW, TPW),
        s1.reshape(NW, TPW),
        w[:, 0].reshape(NW, TPW),
        w[:, 1].reshape(NW, TPW),
    )


# speedup vs baseline: 1.1634x; 1.0253x over previous
import functools

import jax
import jax.numpy as jnp
from jax import lax
from jax.experimental import pallas as pl
from jax.experimental.pallas import tpu as pltpu
from jax.experimental.pallas import tpu_sc as plsc

NE = 16
NT = 4096
NA = 2 * NT
BLK = 256
NBMAX = NA // BLK + NE
CAP = NBMAX * BLK
CHUNK = 512
NCHUNK = NA // CHUNK
NC, NS = 2, 16
NW = NC * NS
TPW = NT // NW
TB = 1024
RPC = 64
NDCH = TPW // RPC
DP = 512
TPC = 16
NCCH = TPW // TPC


def _routing_body(x_ref, wr_ref, br_ref, xp_ref, s0_ref, s1_ref, w_ref,
                  bexp_ref, probs_ref, oh_ref, call_ref, p_ref):
    t = pl.program_id(0)
    xb = x_ref[...]

    def b16top(v):
        tt = lax.bitcast_convert_type(v, jnp.int32)
        r = tt + 0x7FFF + (lax.shift_right_logical(tt, 16) & 1)
        return r & jnp.int32(-65536)

    xp_ref[...] = (
        lax.shift_right_logical(b16top(xb[:, :DP]), 16) | b16top(xb[:, DP:])
    )
    logits = (
        jnp.dot(xb, wr_ref[...], preferred_element_type=jnp.float32)
        + br_ref[...]
    )
    m = jnp.max(logits, axis=-1, keepdims=True)
    ex = jnp.exp(logits - m)
    probs_ref[pl.ds(t * TB, TB), :] = ex / jnp.sum(ex, axis=-1, keepdims=True)

    @pl.when(t == NT // TB - 1)
    def _routing_tail():
        probs = probs_ref[...]
        lane = lax.broadcasted_iota(jnp.int32, probs.shape, 1)
        m1 = jnp.max(probs, axis=-1, keepdims=True)
        a1 = jnp.min(jnp.where(probs == m1, lane, NE), axis=-1, keepdims=True)
        rest = jnp.where(lane == a1, -jnp.inf, probs)
        m2 = jnp.max(rest, axis=-1, keepdims=True)
        a2 = jnp.min(jnp.where(rest == m2, lane, NE), axis=-1, keepdims=True)
        w_ref[:, 0:1] = m1
        w_ref[:, 1:2] = m2
        oh_ref[0:NT] = (lane == a1).astype(jnp.float32)
        oh_ref[NT:NA] = (lane == a2).astype(jnp.float32)

        r_i = lax.broadcasted_iota(jnp.int32, (CHUNK, CHUNK), 0)
        c_i = lax.broadcasted_iota(jnp.int32, (CHUNK, CHUNK), 1)
        tri = (r_i >= c_i).astype(jnp.float32)

        def chunk_cumsum(c, _):
            oc = oh_ref[pl.ds(c * CHUNK, CHUNK), :]
            cc = jnp.dot(tri, oc, preferred_element_type=jnp.float32)
            call_ref[pl.ds(c * CHUNK, CHUNK), :] = cc
            p_ref[pl.ds(c, 1), :] = cc[CHUNK - 1 : CHUNK, :]
            return 0

        lax.fori_loop(0, NCHUNK, chunk_cumsum, 0)

        rn = lax.broadcasted_iota(jnp.int32, (NCHUNK, NCHUNK), 0)
        cn = lax.broadcasted_iota(jnp.int32, (NCHUNK, NCHUNK), 1)
        trin = (rn > cn).astype(jnp.float32)
        totals = p_ref[...]
        pref = jnp.dot(trin, totals, preferred_element_type=jnp.float32)
        counts = pref[NCHUNK - 1 : NCHUNK, :] + totals[NCHUNK - 1 : NCHUNK, :]

        nb = jnp.floor((counts + (BLK - 1)) * (1.0 / BLK))
        le = lax.broadcasted_iota(jnp.int32, (NE, NE), 0)
        lf = lax.broadcasted_iota(jnp.int32, (NE, NE), 1)
        u_excl = (le < lf).astype(jnp.float32)
        u_incl = (le <= lf).astype(jnp.float32)
        start_rows = BLK * jnp.dot(nb, u_excl, preferred_element_type=jnp.float32)
        cum_incl = jnp.dot(nb, u_incl, preferred_element_type=jnp.float32)
        total_blocks = cum_incl[0:1, NE - 1 : NE]
        bi = lax.broadcasted_iota(jnp.int32, (NBMAX, NE), 0).astype(jnp.float32)
        bexp = jnp.sum((cum_incl <= bi).astype(jnp.float32), axis=-1, keepdims=True)
        act = bi[:, 0:1] < total_blocks
        bexp_ref[...] = jnp.where(act, bexp, NE - 1.0).astype(jnp.int32)

        p_ref[...] = pref

        def chunk_slot(c, _):
            oc = oh_ref[pl.ds(c * CHUNK, CHUNK), :]
            r_incl = call_ref[pl.ds(c * CHUNK, CHUNK), :] + p_ref[pl.ds(c, 1), :]
            r_excl = r_incl - oc
            slot = jnp.sum((start_rows + r_excl) * oc, axis=-1, keepdims=True)
            half = c < (NCHUNK // 2)
            slot_i = slot.astype(jnp.int32)

            @pl.when(half)
            def _():
                s0_ref[pl.ds(c * CHUNK, CHUNK), :] = slot_i

            @pl.when(jnp.logical_not(half))
            def _():
                s1_ref[pl.ds(c * CHUNK - NT, CHUNK), :] = slot_i

            return 0

        lax.fori_loop(0, NCHUNK, chunk_slot, 0)


def _routing(x, Wr, br):
    n, d = x.shape
    return pl.pallas_call(
        _routing_body,
        grid=(n // TB,),
        in_specs=[
            pl.BlockSpec((TB, d), lambda t: (t, 0)),
            pl.BlockSpec((d, NE), lambda t: (0, 0)),
            pl.BlockSpec((1, NE), lambda t: (0, 0)),
        ],
        out_specs=[
            pl.BlockSpec((TB, DP), lambda t: (t, 0)),
            pl.BlockSpec((NT, 1), lambda t: (0, 0)),
            pl.BlockSpec((NT, 1), lambda t: (0, 0)),
            pl.BlockSpec((NT, 2), lambda t: (0, 0)),
            pl.BlockSpec((NBMAX, 1), lambda t: (0, 0)),
        ],
        out_shape=[
            jax.ShapeDtypeStruct((NT, DP), jnp.int32),
            jax.ShapeDtypeStruct((NT, 1), jnp.int32),
            jax.ShapeDtypeStruct((NT, 1), jnp.int32),
            jax.ShapeDtypeStruct((NT, 2), jnp.float32),
            jax.ShapeDtypeStruct((NBMAX, 1), jnp.int32),
        ],
        scratch_shapes=[
            pltpu.VMEM((NT, NE), jnp.float32),
            pltpu.VMEM((NA, NE), jnp.float32),
            pltpu.VMEM((NA, NE), jnp.float32),
            pltpu.VMEM((NCHUNK, NE), jnp.float32),
        ],
        compiler_params=pltpu.CompilerParams(dimension_semantics=("arbitrary",)),
    )(x, Wr, br.reshape(1, NE))


def _dispatch_body(
    x_hbm, s0_hbm, s1_hbm, xs_hbm,
    rows_a, rows_b, i0_s, i1_s, sem_ld, sem_st,
):
    wid = lax.axis_index("s") * NC + lax.axis_index("c")
    bufs = (rows_a, rows_b)
    pltpu.sync_copy(s0_hbm.at[wid], i0_s)
    pltpu.sync_copy(s1_hbm.at[wid], i1_s)

    base0 = wid * TPW
    ld = pltpu.async_copy(x_hbm.at[pl.ds(base0, RPC)], rows_a, sem_ld)
    for ch in range(NDCH):
        cur = bufs[ch % 2]
        ld.wait()
        if ch + 1 < NDCH:
            ld = pltpu.async_copy(
                x_hbm.at[pl.ds(base0 + (ch + 1) * RPC, RPC)], bufs[(ch + 1) % 2],
                sem_ld,
            )
        h0 = pltpu.async_copy(cur, xs_hbm.at[i0_s.at[ch]], sem_st)
        h1 = pltpu.async_copy(cur, xs_hbm.at[i1_s.at[ch]], sem_st)
        h0.wait()
        h1.wait()


def _dispatch(xp, s0_3d, s1_3d):
    d = xp.shape[1]
    mesh = plsc.VectorSubcoreMesh(
        core_axis_name="c", subcore_axis_name="s", num_cores=NC, num_subcores=NS
    )
    f = functools.partial(
        pl.kernel,
        out_type=jax.ShapeDtypeStruct((CAP, d), jnp.int32),
        mesh=mesh,
        scratch_types=[
            pltpu.VMEM((RPC, d), jnp.int32),
            pltpu.VMEM((RPC, d), jnp.int32),
            pltpu.VMEM((NDCH, RPC), jnp.int32),
            pltpu.VMEM((NDCH, RPC), jnp.int32),
            pltpu.SemaphoreType.DMA,
            pltpu.SemaphoreType.DMA,
        ],
    )(_dispatch_body)
    return f(xp, s0_3d, s1_3d)


def _gmm_body(bexp_smem, x_ref, we_ref, be_ref, y_ref):
    del bexp_smem
    xw = x_ref[...]
    xlo = lax.bitcast_convert_type(lax.shift_left(xw, 16), jnp.float32)
    xhi = lax.bitcast_convert_type(xw & jnp.int32(-65536), jnp.float32)
    we = we_ref[0]
    y = jnp.dot(xlo, we[:DP, :], preferred_element_type=jnp.float32)
    y += jnp.dot(xhi, we[DP:, :], preferred_element_type=jnp.float32)
    y_ref[...] = jnp.maximum(y + be_ref[0], 0.0)


def _gmm(block_expert, xs, We, be):
    ne, d, u = We.shape
    grid_spec = pltpu.PrefetchScalarGridSpec(
        num_scalar_prefetch=1,
        grid=(NBMAX,),
        in_specs=[
            pl.BlockSpec((BLK, DP), lambda b, bexp: (b, 0)),
            pl.BlockSpec((1, d, u), lambda b, bexp: (bexp[b], 0, 0)),
            pl.BlockSpec((1, 1, u), lambda b, bexp: (bexp[b], 0, 0)),
        ],
        out_specs=pl.BlockSpec((BLK, u), lambda b, bexp: (b, 0)),
    )
    return pl.pallas_call(
        _gmm_body,
        grid_spec=grid_spec,
        out_shape=jax.ShapeDtypeStruct((CAP, u), jnp.float32),
        compiler_params=pltpu.CompilerParams(dimension_semantics=("arbitrary",)),
    )(block_expert, xs, We, be.reshape(ne, 1, u))


def _combine_body(
    y_hbm, s0_hbm, s1_hbm, w0_hbm, w1_hbm, out_hbm,
    r0_a, r1_a, o_a, r0_b, r1_b, o_b, i0_all, i1_all, w0_all, w1_all,
    sem_a, sem_b, sem_o,
):
    wid = lax.axis_index("s") * NC + lax.axis_index("c")
    r0s = (r0_a, r0_b)
    r1s = (r1_a, r1_b)
    os_ = (o_a, o_b)
    sems = (sem_a, sem_b)
    pltpu.sync_copy(s0_hbm.at[wid], i0_all)
    pltpu.sync_copy(s1_hbm.at[wid], i1_all)
    pltpu.sync_copy(w0_hbm.at[wid], w0_all)
    pltpu.sync_copy(w1_hbm.at[wid], w1_all)

    def issue(ch):
        k = ch % 2
        idx0 = i0_all[pl.ds(ch * TPC, TPC)]
        idx1 = i1_all[pl.ds(ch * TPC, TPC)]
        g0 = pltpu.async_copy(y_hbm.at[idx0], r0s[k], sems[k])
        g1 = pltpu.async_copy(y_hbm.at[idx1], r1s[k], sems[k])
        return (g0, g1)

    pend = issue(0)
    st = None
    for ch in range(NCCH):
        k = ch % 2
        pend[0].wait()
        pend[1].wait()
        if ch + 1 < NCCH:
            pend = issue(ch + 1)
        if st is not None:
            st.wait()
        r0, r1, o = r0s[k], r1s[k], os_[k]
        wv0 = w0_all[pl.ds(ch * TPC, TPC)]
        wv1 = w1_all[pl.ds(ch * TPC, TPC)]

        def tok(i, _):
            i_vec = lax.broadcast_in_dim(i, (16,), ())
            wa = wv0.at[i_vec].get(mode="promise_in_bounds")
            wb = wv1.at[i_vec].get(mode="promise_in_bounds")

            def vec(v, _):
                sl = pl.ds(v * 16, 16)
                o[i, sl] = wa * r0[i, sl] + wb * r1[i, sl]
                return 0

            lax.fori_loop(0, 1024 // 16, vec, 0, unroll=8)
            return 0

        lax.fori_loop(0, TPC, tok, 0)
        base = wid * TPW + ch * TPC
        st = pltpu.async_copy(o, out_hbm.at[pl.ds(base, TPC)], sem_o)
    st.wait()


def _combine(y, s0_2d, s1_2d, w0_2d, w1_2d):
    u = y.shape[1]
    mesh = plsc.VectorSubcoreMesh(
        core_axis_name="c", subcore_axis_name="s", num_cores=NC, num_subcores=NS
    )
    f = functools.partial(
        pl.kernel,
        out_type=jax.ShapeDtypeStruct((NT, u), jnp.float32),
        mesh=mesh,
        scratch_types=[
            pltpu.VMEM((TPC, u), jnp.float32),
            pltpu.VMEM((TPC, u), jnp.float32),
            pltpu.VMEM((TPC, u), jnp.float32),
            pltpu.VMEM((TPC, u), jnp.float32),
            pltpu.VMEM((TPC, u), jnp.float32),
            pltpu.VMEM((TPC, u), jnp.float32),
            pltpu.VMEM((TPW,), jnp.int32),
            pltpu.VMEM((TPW,), jnp.int32),
            pltpu.VMEM((TPW,), jnp.float32),
            pltpu.VMEM((TPW,), jnp.float32),
            pltpu.SemaphoreType.DMA,
            pltpu.SemaphoreType.DMA,
            pltpu.SemaphoreType.DMA,
        ],
    )(_combine_body)
    return f(y, s0_2d, s1_2d, w0_2d, w1_2d)


def kernel(inputs, Wr, br, We, be):
    xp, s0c, s1c, w, bexpc = _routing(inputs, Wr, br)
    s0 = s0c.reshape(NT)
    s1 = s1c.reshape(NT)
    xs = _dispatch(xp, s0.reshape(NW, NDCH, RPC), s1.reshape(NW, NDCH, RPC))
    y = _gmm(bexpc[:, 0], xs, We, be)
    return _combine(
        y,
        s0.reshape(NW, TPW),
        s1.reshape(NW, TPW),
        w[:, 0].reshape(NW, TPW),
        w[:, 1].reshape(NW, TPW),
    )
